# baseline (speedup 1.0000x reference)
import math

import jax
import jax.numpy as jnp
from jax import lax
from jax.experimental import pallas as pl
from jax.experimental.pallas import tpu as pltpu
from jax.experimental.pallas import tpu_sc as plsc

VOCAB = 1000000
EMB = 64
B = 4096
L = 200
SCALE = math.sqrt(EMB)

NC, NS = 2, 16
NW = NC * NS
BB = B // NW
SLAB = EMB * BB
TROW = SLAB // 8


def _emb_kernel(tok_hbm, table_hbm, out_hbm, idx_v, h0, h1, f0, f1, g0,
                g1, t0, t1, gsem0, gsem1, osem0, osem1):
    wid = lax.axis_index("s") * NC + lax.axis_index("c")
    h_p = (h0, h1)
    f_p = (f0, f1)
    g_p = (g0, g1)
    t_p = (t0, t1)
    gsem_p = (gsem0, gsem1)
    osem_p = (osem0, osem1)

    iota = lax.iota(jnp.int32, 16)
    evec = [iota + 16 * cc for cc in range(EMB // 16)]
    trow = [(iota // 8) + 2 * cc for cc in range(EMB // 16)]
    srow = iota % 8

    pltpu.sync_copy(tok_hbm.at[:, wid, :, :], idx_v)

    def prep_idx(c, p):
        @plsc.parallel_loop(0, BB // 16, unroll=8)
        def _(j):
            v = plsc.load_gather(
                idx_v, [jnp.full((16,), c // 8, jnp.int32),
                        jnp.full((16,), c % 8, jnp.int32),
                        iota + j * 16])
            h_p[p][pl.ds(j * 16, 16)] = (v & 1) * EMB
            f_p[p][pl.ds(j * 16, 16)] = v >> 1

    def fire_gather(c, p):
        pltpu.async_copy(table_hbm.at[f_p[p]], g_p[p], gsem_p[p])

    def drain_gather(p):
        pltpu.make_async_copy(table_hbm.at[f_p[p]], g_p[p],
                              gsem_p[p]).wait()

    def fire_out(c, p):
        pltpu.async_copy(t_p[p], out_hbm.at[c, :, wid, :, :], osem_p[p])

    def drain_out(p):
        pltpu.make_async_copy(t_p[p], out_hbm.at[0, :, 0, :, :],
                              osem_p[p]).wait()

    prep_idx(0, 0)
    fire_gather(0, 0)
    prep_idx(1, 1)
    fire_gather(1, 1)

    def pair_body(i, _):
        for p in range(2):
            c = 2 * i + p
            drain_gather(p)

            @pl.when(i >= 1)
            def _():
                drain_out(p)

            @plsc.parallel_loop(0, BB, unroll=8, carry=iota)
            def _(k, mvec):
                hv = plsc.load_gather(h_p[p], [mvec])
                for cc in range(EMB // 16):
                    x = plsc.load_gather(
                        g_p[p], [mvec, hv + evec[cc]]) * SCALE
                    plsc.store_scatter(t_p[p], [trow[cc], srow, mvec], x)
                return (mvec + 1) & 127
            fire_out(c, p)

            @pl.when(c + 2 < L)
            def _():
                prep_idx(c + 2, p)
                fire_gather(c + 2, p)

        return 0

    lax.fori_loop(0, L // 2, pair_body, 0)
    drain_out(0)
    drain_out(1)


@jax.jit
def kernel(tokens, table):
    mesh = plsc.VectorSubcoreMesh(core_axis_name="c", subcore_axis_name="s")
    tok_tiles = (tokens.astype(jnp.int32)
                 .reshape(NW, BB, L // 8, 8)
                 .transpose(2, 0, 3, 1))
    out4 = pl.kernel(
        _emb_kernel,
        out_type=jax.ShapeDtypeStruct((L, 8, NW, 8, BB), jnp.float32),
        mesh=mesh,
        scratch_types=[
            pltpu.VMEM((L // 8, 8, BB), jnp.int32),
            pltpu.VMEM((BB,), jnp.int32),
            pltpu.VMEM((BB,), jnp.int32),
            pltpu.VMEM((BB,), jnp.int32),
            pltpu.VMEM((BB,), jnp.int32),
            pltpu.VMEM((BB, 2 * EMB), jnp.float32),
            pltpu.VMEM((BB, 2 * EMB), jnp.float32),
            pltpu.VMEM((8, 8, BB), jnp.float32),
            pltpu.VMEM((8, 8, BB), jnp.float32),
            pltpu.SemaphoreType.DMA,
            pltpu.SemaphoreType.DMA,
            pltpu.SemaphoreType.DMA,
            pltpu.SemaphoreType.DMA,
        ],
        compiler_params=pltpu.CompilerParams(use_tc_tiling_on_sc=True,
                                             needs_layout_passes=False),
    )(tok_tiles, table.reshape(VOCAB // 2, 2 * EMB))
    return out4.transpose(2, 4, 0, 1, 3).reshape(B, L, EMB)


# speedup vs baseline: 1.1215x; 1.1215x over previous
import math

import jax
import jax.numpy as jnp
from jax import lax
from jax.experimental import pallas as pl
from jax.experimental.pallas import tpu as pltpu
from jax.experimental.pallas import tpu_sc as plsc

VOCAB = 1000000
EMB = 64
B = 4096
L = 200
SCALE = math.sqrt(EMB)

NC, NS = 2, 16
NW = NC * NS
BB = B // NW
SLAB = EMB * BB
TROW = SLAB // 8


def _emb_kernel(tok_hbm, table_hbm, out_hbm, idx_v, g0, g1, g2,
                t0, t1, gsem0, gsem1, gsem2, osem0, osem1):
    wid = lax.axis_index("s") * NC + lax.axis_index("c")
    g_p = (g0, g1, g2)
    t_p = (t0, t1)
    gsem_p = (gsem0, gsem1, gsem2)
    osem_p = (osem0, osem1)

    iota = lax.iota(jnp.int32, 16)
    evec = [iota + 16 * cc for cc in range(EMB // 16)]
    trow = [(iota // 8) + 2 * cc for cc in range(EMB // 16)]
    tcol = (iota % 8) * BB

    pltpu.sync_copy(tok_hbm.at[:, wid, :], idx_v)

    def fire_gather(c, p):
        pltpu.async_copy(
            table_hbm.at[idx_v.at[c // 8, pl.ds((c % 8) * BB, BB)]],
            g_p[p], gsem_p[p])

    def drain_gather(p):
        pltpu.make_async_copy(
            table_hbm.at[idx_v.at[0, pl.ds(0, BB)]], g_p[p],
            gsem_p[p]).wait()

    def fire_out(c, p):
        pltpu.async_copy(t_p[p], out_hbm.at[c, :, wid], osem_p[p])

    def drain_out(p):
        pltpu.make_async_copy(t_p[p], out_hbm.at[0, :, 0],
                              osem_p[p]).wait()

    fire_gather(0, 0)
    fire_gather(1, 1)
    fire_gather(2, 2)

    def six_body(i, _):
        for k in range(6):
            c = 6 * i + k
            g = k % 3
            p = k % 2
            drain_gather(g)

            if k < 2:
                @pl.when(i >= 1)
                def _():
                    drain_out(p)
            else:
                drain_out(p)

            @plsc.parallel_loop(0, BB, unroll=8, carry=iota)
            def _(kk, mvec):
                for cc in range(EMB // 16):
                    x = plsc.load_gather(g_p[g], [mvec, evec[cc]]) * SCALE
                    plsc.store_scatter(t_p[p], [trow[cc], tcol + mvec], x)
                return (mvec + 1) & 127
            fire_out(c, p)

            @pl.when(c + 3 < L)
            def _():
                fire_gather(c + 3, g)

        return 0

    lax.fori_loop(0, (L - 2) // 6, six_body, 0)
    for c in (L - 2, L - 1):
        g = c % 3
        p = c % 2
        drain_gather(g)
        drain_out(p)

        @plsc.parallel_loop(0, BB, unroll=8, carry=iota)
        def _(kk, mvec, g=g, p=p):
            for cc in range(EMB // 16):
                x = plsc.load_gather(g_p[g], [mvec, evec[cc]]) * SCALE
                plsc.store_scatter(t_p[p], [trow[cc], tcol + mvec], x)
            return (mvec + 1) & 127

        fire_out(c, p)
    drain_out(0)
    drain_out(1)


@jax.jit
def kernel(tokens, table):
    mesh = plsc.VectorSubcoreMesh(core_axis_name="c", subcore_axis_name="s")
    tok_tiles = (tokens.astype(jnp.int32)
                 .reshape(NW, BB, L // 8, 8)
                 .transpose(2, 0, 3, 1)
                 .reshape(L // 8, NW, 8 * BB))
    out4 = pl.kernel(
        _emb_kernel,
        out_type=jax.ShapeDtypeStruct((L, 8, NW, TROW), jnp.float32),
        mesh=mesh,
        scratch_types=[
            pltpu.VMEM((L // 8, 8 * BB), jnp.int32),
            pltpu.VMEM((BB, EMB), jnp.float32),
            pltpu.VMEM((BB, EMB), jnp.float32),
            pltpu.VMEM((BB, EMB), jnp.float32),
            pltpu.VMEM((8, TROW), jnp.float32),
            pltpu.VMEM((8, TROW), jnp.float32),
            pltpu.SemaphoreType.DMA,
            pltpu.SemaphoreType.DMA,
            pltpu.SemaphoreType.DMA,
            pltpu.SemaphoreType.DMA,
            pltpu.SemaphoreType.DMA,
        ],
        compiler_params=pltpu.CompilerParams(use_tc_tiling_on_sc=False,
                                             needs_layout_passes=False),
    )(tok_tiles, table)
    r5 = out4.reshape(L, 8, NW, 8, BB)
    return r5.transpose(2, 4, 0, 1, 3).reshape(B, L, EMB)
